# Initial kernel scaffold; baseline (speedup 1.0000x reference)
import jax
import jax.numpy as jnp
from jax.experimental import pallas as pl


def kernel(x, table):
    raise NotImplementedError("write your pallas kernel here")


# speedup vs baseline: 5.2998x; 5.2998x over previous
import functools

import jax
import jax.numpy as jnp
from jax import lax
from jax.experimental import pallas as pl
from jax.experimental.pallas import tpu as pltpu
from jax.experimental.pallas import tpu_sc as plsc

NC = 2
NS = 16
NW = NC * NS
CHUNK = 128
G = 5


def _sc_gather(x3, table):
    _, n_chunks, _ = x3.shape
    D = table.shape[1]
    T = n_chunks // G
    assert n_chunks % G == 0 and T % 2 == 0

    mesh = plsc.VectorSubcoreMesh(core_axis_name="c", subcore_axis_name="s")

    @functools.partial(
        pl.kernel,
        out_type=jax.ShapeDtypeStruct((NW * n_chunks, CHUNK, D), jnp.float32),
        mesh=mesh,
        compiler_params=pltpu.CompilerParams(use_tc_tiling_on_sc=False),
        scratch_types=[
            pltpu.VMEM((n_chunks, CHUNK), jnp.int32),
            pltpu.VMEM((G, CHUNK, D), jnp.float32),
            pltpu.VMEM((G, CHUNK, D), jnp.float32),
            pltpu.SemaphoreType.DMA,
            pltpu.SemaphoreType.DMA,
            pltpu.SemaphoreType.DMA,
            pltpu.SemaphoreType.DMA,
        ],
    )
    def k(x_hbm, table_hbm, out_hbm, idx_v, buf0, buf1, g0, g1, s0, s1):
        wid = lax.axis_index("s") * NC + lax.axis_index("c")
        chunk0 = wid * n_chunks

        pltpu.sync_copy(x_hbm.at[wid], idx_v)

        def fire_gathers(t, buf, gsem):
            for b in range(G):
                pltpu.async_copy(table_hbm.at[idx_v.at[t * G + b]], buf.at[b], gsem)

        def drain_gathers(t, buf, gsem):
            for b in range(G):
                pltpu.make_async_copy(
                    table_hbm.at[idx_v.at[t * G + b]], buf.at[b], gsem
                ).wait()

        def fire_scatter(t, buf, ssem):
            pltpu.async_copy(buf, out_hbm.at[pl.ds(chunk0 + t * G, G)], ssem)

        def drain_scatter(t, buf, ssem):
            pltpu.make_async_copy(
                buf, out_hbm.at[pl.ds(chunk0 + t * G, G)], ssem
            ).wait()

        fire_gathers(0, buf0, g0)

        @pl.loop(0, T, step=2)
        def _(t2):
            @pl.when(t2 > 0)
            def _():
                drain_scatter(t2 - 1, buf1, s1)

            fire_gathers(t2 + 1, buf1, g1)
            drain_gathers(t2, buf0, g0)
            fire_scatter(t2, buf0, s0)

            @pl.when(t2 < T - 2)
            def _():
                drain_scatter(t2, buf0, s0)
                fire_gathers(t2 + 2, buf0, g0)

            drain_gathers(t2 + 1, buf1, g1)
            fire_scatter(t2 + 1, buf1, s1)

        drain_scatter(T - 2, buf0, s0)
        drain_scatter(T - 1, buf1, s1)

    return k(x3, table)


def kernel(x, table):
    B0, B1 = x.shape
    D = table.shape[1]
    n_chunks = (B0 * B1) // (NW * CHUNK)
    x3 = x.reshape(NW, n_chunks, CHUNK)
    out = _sc_gather(x3, table)
    return out.reshape(B0, B1, D)


# speedup vs baseline: 5.3071x; 1.0014x over previous
import functools

import jax
import jax.numpy as jnp
from jax import lax
from jax.experimental import pallas as pl
from jax.experimental.pallas import tpu as pltpu
from jax.experimental.pallas import tpu_sc as plsc

NC = 2
NS = 16
NW = NC * NS
CHUNK = 128
G = 10


def _sc_gather(x3, table):
    _, n_chunks, _ = x3.shape
    D = table.shape[1]
    T = n_chunks // G
    assert n_chunks % G == 0 and T % 2 == 0

    mesh = plsc.VectorSubcoreMesh(core_axis_name="c", subcore_axis_name="s")

    @functools.partial(
        pl.kernel,
        out_type=jax.ShapeDtypeStruct((NW * n_chunks, CHUNK, D), jnp.float32),
        mesh=mesh,
        compiler_params=pltpu.CompilerParams(use_tc_tiling_on_sc=False),
        scratch_types=[
            pltpu.VMEM((n_chunks, CHUNK), jnp.int32),
            pltpu.VMEM((G, CHUNK, D), jnp.float32),
            pltpu.VMEM((G, CHUNK, D), jnp.float32),
            pltpu.SemaphoreType.DMA,
            pltpu.SemaphoreType.DMA,
            pltpu.SemaphoreType.DMA,
            pltpu.SemaphoreType.DMA,
        ],
    )
    def k(x_hbm, table_hbm, out_hbm, idx_v, buf0, buf1, g0, g1, s0, s1):
        wid = lax.axis_index("s") * NC + lax.axis_index("c")
        chunk0 = wid * n_chunks

        pltpu.sync_copy(x_hbm.at[wid], idx_v)

        def fire_gathers(t, buf, gsem):
            for b in range(G):
                pltpu.async_copy(table_hbm.at[idx_v.at[t * G + b]], buf.at[b], gsem)

        def drain_gathers(t, buf, gsem):
            for b in range(G):
                pltpu.make_async_copy(
                    table_hbm.at[idx_v.at[t * G + b]], buf.at[b], gsem
                ).wait()

        def fire_scatter(t, buf, ssem):
            pltpu.async_copy(buf, out_hbm.at[pl.ds(chunk0 + t * G, G)], ssem)

        def drain_scatter(t, buf, ssem):
            pltpu.make_async_copy(
                buf, out_hbm.at[pl.ds(chunk0 + t * G, G)], ssem
            ).wait()

        fire_gathers(0, buf0, g0)

        @pl.loop(0, T, step=2)
        def _(t2):
            @pl.when(t2 > 0)
            def _():
                drain_scatter(t2 - 1, buf1, s1)

            fire_gathers(t2 + 1, buf1, g1)
            drain_gathers(t2, buf0, g0)
            fire_scatter(t2, buf0, s0)

            @pl.when(t2 < T - 2)
            def _():
                drain_scatter(t2, buf0, s0)
                fire_gathers(t2 + 2, buf0, g0)

            drain_gathers(t2 + 1, buf1, g1)
            fire_scatter(t2 + 1, buf1, s1)

        drain_scatter(T - 2, buf0, s0)
        drain_scatter(T - 1, buf1, s1)

    return k(x3, table)


def kernel(x, table):
    B0, B1 = x.shape
    D = table.shape[1]
    n_chunks = (B0 * B1) // (NW * CHUNK)
    x3 = x.reshape(NW, n_chunks, CHUNK)
    out = _sc_gather(x3, table)
    return out.reshape(B0, B1, D)


# speedup vs baseline: 7.4199x; 1.3981x over previous
import functools

import jax
import jax.numpy as jnp
from jax import lax
from jax.experimental import pallas as pl
from jax.experimental.pallas import tpu as pltpu
from jax.experimental.pallas import tpu_sc as plsc

NC = 2
NS = 16
NW = NC * NS


def _sc_gather_t(xT, tT):
    P, B = xT.shape
    D, V = tT.shape
    L = 16

    mesh = plsc.VectorSubcoreMesh(core_axis_name="c", subcore_axis_name="s")

    @functools.partial(
        pl.kernel,
        out_type=jax.ShapeDtypeStruct((P, D, B), jnp.float32),
        mesh=mesh,
        compiler_params=pltpu.CompilerParams(
            use_tc_tiling_on_sc=True, needs_layout_passes=False
        ),
        scratch_types=[
            pltpu.VMEM((V,), jnp.float32),
            pltpu.VMEM((2, B), jnp.int32),
            pltpu.VMEM((2, B), jnp.float32),
            pltpu.SemaphoreType.DMA,
            pltpu.SemaphoreType.DMA,
            pltpu.SemaphoreType.DMA,
        ],
    )
    def k(xT_hbm, tT_hbm, out_hbm, trow, idxb, outb, tsem, isem, osem):
        f = lax.axis_index("s") * NC + lax.axis_index("c")
        row_cp = pltpu.async_copy(tT_hbm.at[f], trow, tsem)

        def fire_idx(p, slot):
            pltpu.async_copy(xT_hbm.at[p], idxb.at[slot], isem)

        def drain_idx(p, slot):
            pltpu.make_async_copy(xT_hbm.at[p], idxb.at[slot], isem).wait()

        def fire_out(p, slot):
            pltpu.async_copy(outb.at[slot], out_hbm.at[p, f], osem)

        def drain_out(p, slot):
            pltpu.make_async_copy(outb.at[slot], out_hbm.at[p, f], osem).wait()

        fire_idx(0, 0)
        row_cp.wait()

        def compute(islot, oslot):
            @pl.loop(0, B // L)
            def _(i):
                idx = idxb[islot, pl.ds(i * L, L)]
                outb[oslot, pl.ds(i * L, L)] = plsc.load_gather(trow, [idx])

        @pl.loop(0, P, step=2)
        def _(p2):
            fire_idx(p2 + 1, 1)
            drain_idx(p2, 0)
            compute(0, 0)

            @pl.when(p2 > 0)
            def _():
                drain_out(p2 - 1, 1)

            fire_out(p2, 0)

            @pl.when(p2 < P - 2)
            def _():
                fire_idx(p2 + 2, 0)

            drain_idx(p2 + 1, 1)
            compute(1, 1)
            drain_out(p2, 0)
            fire_out(p2 + 1, 1)

        drain_out(P - 1, 1)

    return k(xT, tT)


def kernel(x, table):
    B0, B1 = x.shape
    D = table.shape[1]
    outT = _sc_gather_t(x.T, table.T)
    return outT.transpose(2, 0, 1)
